# baseline (speedup 1.0000x reference)
import functools

import jax
import jax.numpy as jnp
from jax import lax
from jax.experimental import pallas as pl
from jax.experimental.pallas import tpu as pltpu
from jax.experimental.pallas import tpu_sc as plsc

N, S, ARG_NUM = 1024, 64, 16
B = N * S
D = 128
E = 64
ROW_STRIDE = 264
MRG_ROWS = 6 * ROW_STRIDE
TBL_ROWS = MRG_ROWS + 15 * ROW_STRIDE
NC, NS = 2, 16
NW = NC * NS
CHUNK = 128
NCHUNK = (B // NW) // CHUNK
TOK_PER_W = B // NW
NSP = 13


def _fold_body(cmd_ref, arg_ref, w_ref, b_ref, out_ref):
    a = arg_ref[...]
    t0 = jnp.dot(a, w_ref[pl.ds(0, E), :],
                 preferred_element_type=jnp.float32)
    cb = cmd_ref[...] + b_ref[...]
    for c in range(6):
        out_ref[pl.ds(c * ROW_STRIDE, 257), :] = t0 + cb[c:c + 1, :]
    for k in range(1, ARG_NUM):
        wk = w_ref[pl.ds(k * E, E), :]
        out_ref[pl.ds(MRG_ROWS + (k - 1) * ROW_STRIDE, 257), :] = jnp.dot(
            a, wk, preferred_element_type=jnp.float32)


def _fold(cmd_table, arg_table, W, b2):
    return pl.pallas_call(
        _fold_body,
        out_shape=jax.ShapeDtypeStruct((TBL_ROWS, D), jnp.float32),
    )(cmd_table, arg_table, W, b2)


def _sc_body(table_hbm, cmdf, argsT, out, rawc, rawa, idx, acc, acc2, table,
             sg0, sg1, sh0, sh1, sr0, sr1, sra0, sra1, so0, so1):
    sg = [sg0, sg1]
    sh = [sh0, sh1]
    sr = [sr0, sr1]
    sra = [sra0, sra1]
    so = [so0, so1]
    sid = lax.axis_index("s")
    wid = sid * NC + lax.axis_index("c")
    base0 = wid * TOK_PER_W

    @pl.when(sid == 0)
    def _stage():
        pltpu.sync_copy(table_hbm, table)
    plsc.subcore_barrier()

    def wait_raw(b, base):
        pltpu.make_async_copy(cmdf.at[pl.ds(base, CHUNK)],
                              rawc.at[b], sr[b]).wait()
        pltpu.make_async_copy(argsT.at[:, pl.ds(base, CHUNK)],
                              rawa.at[b], sra[b]).wait()

    def fire_raw(b, base):
        pltpu.async_copy(cmdf.at[pl.ds(base, CHUNK)], rawc.at[b], sr[b])
        pltpu.async_copy(argsT.at[:, pl.ds(base, CHUNK)], rawa.at[b], sra[b])

    def adjust(b):
        def body(g8, carry):
            t0 = g8 * 16
            idx[b, 0, pl.ds(t0, 16)] = (
                rawc[b, pl.ds(t0, 16)] * ROW_STRIDE
                + rawa[b, 0, pl.ds(t0, 16)] + 1)
            for s in range(1, ARG_NUM):
                idx[b, s, pl.ds(t0, 16)] = (
                    rawa[b, s, pl.ds(t0, 16)]
                    + (MRG_ROWS + (s - 1) * ROW_STRIDE + 1))
            return carry
        lax.fori_loop(0, 8, body, 0)

    def zero_accs(b):
        z = jnp.zeros((16,), jnp.float32)

        def body(r, carry):
            for j in range(D // 16):
                acc[b, r, pl.ds(j * 16, 16)] = z
                acc2[b, r, pl.ds(j * 16, 16)] = z
            return carry
        lax.fori_loop(0, CHUNK, body, 0)

    def fire_gathers(b):
        for k in range(NSP):
            pltpu.async_copy(table.at[idx.at[b, k]], acc.at[b],
                             sg[b], add=True)
        for k in range(NSP, ARG_NUM):
            pltpu.async_copy(table_hbm.at[idx.at[b, k]], acc2.at[b],
                             sh[b], add=True)

    def drain_gathers(b):
        for _ in range(NSP):
            pltpu.make_async_copy(table.at[idx.at[b, 0]], acc.at[b],
                                  sg[b]).wait()
        for _ in range(ARG_NUM - NSP):
            pltpu.make_async_copy(table_hbm.at[idx.at[b, 0]], acc2.at[b],
                                  sh[b]).wait()

    def merge_accs(b):
        def body(r, carry):
            for j in range(D // 16):
                sl = pl.ds(j * 16, 16)
                acc[b, r, sl] = acc[b, r, sl] + acc2[b, r, sl]
            return carry
        lax.fori_loop(0, CHUNK, body, 0)

    def fire_out(b, base):
        pltpu.async_copy(acc.at[b], out.at[pl.ds(base, CHUNK), :], so[b])

    def wait_out(b):
        pltpu.make_async_copy(acc.at[b], out.at[pl.ds(0, CHUNK), :],
                              so[b]).wait()

    def chunk(b, base, first=False, second=False, fire_next=True):
        wait_raw(b, base)
        adjust(b)
        if fire_next:
            fire_raw(1 - b, base + CHUNK)
        if not (first or second):
            wait_out(b)
        zero_accs(b)
        fire_gathers(b)
        if not first:
            drain_gathers(1 - b)
            merge_accs(1 - b)
            fire_out(1 - b, base - CHUNK)

    fire_raw(0, base0)
    chunk(0, base0, first=True)
    chunk(1, base0 + CHUNK, second=True)

    def pair(p, carry):
        base = base0 + 2 * p * CHUNK
        chunk(0, base)
        chunk(1, base + CHUNK)
        return carry
    lax.fori_loop(1, NCHUNK // 2 - 1, pair, 0)

    baseL = base0 + (NCHUNK - 2) * CHUNK
    chunk(0, baseL)
    chunk(1, baseL + CHUNK, fire_next=False)
    drain_gathers(1)
    merge_accs(1)
    fire_out(1, baseL + CHUNK)
    wait_out(0)
    wait_out(1)


_sc_lookup = functools.partial(
    pl.kernel,
    out_type=jax.ShapeDtypeStruct((B, D), jnp.float32),
    mesh=plsc.VectorSubcoreMesh(core_axis_name="c", subcore_axis_name="s"),
    scratch_types=[
        pltpu.VMEM((2, CHUNK), jnp.int32),
        pltpu.VMEM((2, ARG_NUM, CHUNK), jnp.int32),
        pltpu.VMEM((2, ARG_NUM, CHUNK), jnp.int32),
        pltpu.VMEM((2, CHUNK, D), jnp.float32),
        pltpu.VMEM((2, CHUNK, D), jnp.float32),
        pltpu.VMEM_SHARED((TBL_ROWS, D), jnp.float32),
        pltpu.SemaphoreType.DMA,
        pltpu.SemaphoreType.DMA,
        pltpu.SemaphoreType.DMA,
        pltpu.SemaphoreType.DMA,
        pltpu.SemaphoreType.DMA,
        pltpu.SemaphoreType.DMA,
        pltpu.SemaphoreType.DMA,
        pltpu.SemaphoreType.DMA,
        pltpu.SemaphoreType.DMA,
        pltpu.SemaphoreType.DMA,
    ],
)(_sc_body)


def kernel(commands, args, cmd_table, arg_table, W, b):
    flat_table = _fold(cmd_table, arg_table, W, b.reshape(1, D))
    argsT = args.reshape(B, ARG_NUM).T
    out = _sc_lookup(flat_table, commands.reshape(B), argsT)
    return out.reshape(N, S, D)


# speedup vs baseline: 1.0144x; 1.0144x over previous
import functools

import jax
import jax.numpy as jnp
from jax import lax
from jax.experimental import pallas as pl
from jax.experimental.pallas import tpu as pltpu
from jax.experimental.pallas import tpu_sc as plsc

N, S, ARG_NUM = 1024, 64, 16
B = N * S
D = 128
E = 64
ROW_STRIDE = 264
MRG_ROWS = 6 * ROW_STRIDE
TBL_ROWS = 5632
NC, NS = 2, 16
NW = NC * NS
CHUNK = 256
G = CHUNK // 128
NCHUNK = (B // NW) // CHUNK
TOK_PER_W = B // NW


def _fold_body(cmd_ref, arg_ref, w_ref, b_ref, out_ref):
    a = arg_ref[...]
    t0 = jnp.dot(a, w_ref[pl.ds(0, E), :],
                 preferred_element_type=jnp.float32)
    cb = cmd_ref[...] + b_ref[...]
    for c in range(6):
        out_ref[pl.ds(c * ROW_STRIDE, 257), :] = t0 + cb[c:c + 1, :]
    for k in range(1, ARG_NUM):
        wk = w_ref[pl.ds(k * E, E), :]
        out_ref[pl.ds(MRG_ROWS + (k - 1) * ROW_STRIDE, 257), :] = jnp.dot(
            a, wk, preferred_element_type=jnp.float32)


def _fold(cmd_table, arg_table, W, b2):
    return pl.pallas_call(
        _fold_body,
        out_shape=jax.ShapeDtypeStruct((TBL_ROWS, D), jnp.float32),
    )(cmd_table, arg_table, W, b2)


def _sc_body(table_hbm, cmdf, argsT, out, rawc, rawa, idx, acc, table,
             sg0, sg1, sr0, sr1, sra0, sra1, so0, so1):
    sg = [sg0, sg1]
    sr = [sr0, sr1]
    sra = [sra0, sra1]
    so = [so0, so1]
    sid = lax.axis_index("s")
    wid = sid * NC + lax.axis_index("c")
    base0 = wid * TOK_PER_W

    def stage_table():
        o = pl.multiple_of(sid * (TBL_ROWS // NS), 8)
        pltpu.sync_copy(table_hbm.at[pl.ds(o, TBL_ROWS // NS), :],
                        table.at[pl.ds(o, TBL_ROWS // NS), :])

    def wait_raw(b, base):
        pltpu.make_async_copy(cmdf.at[pl.ds(base, CHUNK)],
                              rawc.at[b], sr[b]).wait()
        pltpu.make_async_copy(argsT.at[:, pl.ds(base, CHUNK)],
                              rawa.at[b], sra[b]).wait()

    def fire_raw(b, base):
        pltpu.async_copy(cmdf.at[pl.ds(base, CHUNK)], rawc.at[b], sr[b])
        pltpu.async_copy(argsT.at[:, pl.ds(base, CHUNK)], rawa.at[b], sra[b])

    def adjust(b):
        def body(g8, carry):
            for j in range(G):
                t0 = j * 128 + g8 * 16
                idx[b, j, pl.ds(g8 * 16, 16)] = (
                    rawc[b, pl.ds(t0, 16)] * ROW_STRIDE
                    + rawa[b, 0, pl.ds(t0, 16)] + 1)
                for s in range(1, ARG_NUM):
                    idx[b, G * s + j, pl.ds(g8 * 16, 16)] = (
                        rawa[b, s, pl.ds(t0, 16)]
                        + (MRG_ROWS + (s - 1) * ROW_STRIDE + 1))
            return carry
        lax.fori_loop(0, 8, body, 0)

    def zero_acc(b):
        z = jnp.zeros((16,), jnp.float32)

        def body(r, carry):
            for j in range(D // 16):
                acc[b, r, pl.ds(j * 16, 16)] = z
            return carry
        lax.fori_loop(0, CHUNK, body, 0)

    def fire_gathers(b):
        for k in range(ARG_NUM):
            for j in range(G):
                pltpu.async_copy(table.at[idx.at[b, G * k + j]],
                                 acc.at[b, pl.ds(j * 128, 128), :],
                                 sg[b], add=True)

    def drain_gathers(b):
        for _ in range(ARG_NUM * G):
            pltpu.make_async_copy(table.at[idx.at[b, 0]],
                                  acc.at[b, pl.ds(0, 128), :], sg[b]).wait()

    def fire_out(b, base):
        pltpu.async_copy(acc.at[b], out.at[pl.ds(base, CHUNK), :], so[b])

    def wait_out(b):
        pltpu.make_async_copy(acc.at[b], out.at[pl.ds(0, CHUNK), :],
                              so[b]).wait()

    def chunk(b, base, first=False, second=False, fire_next=True):
        wait_raw(b, base)
        adjust(b)
        if fire_next:
            fire_raw(1 - b, base + CHUNK)
        if not (first or second):
            wait_out(b)
        zero_acc(b)
        fire_gathers(b)
        if not first:
            drain_gathers(1 - b)
            fire_out(1 - b, base - CHUNK)

    fire_raw(0, base0)
    fire_raw(1, base0 + CHUNK)
    stage_table()
    plsc.subcore_barrier()
    chunk(0, base0, first=True, fire_next=False)
    chunk(1, base0 + CHUNK, second=True)

    def pair(p, carry):
        base = base0 + 2 * p * CHUNK
        chunk(0, base)
        chunk(1, base + CHUNK)
        return carry
    lax.fori_loop(1, NCHUNK // 2 - 1, pair, 0)

    baseL = base0 + (NCHUNK - 2) * CHUNK
    chunk(0, baseL)
    chunk(1, baseL + CHUNK, fire_next=False)
    drain_gathers(1)
    fire_out(1, baseL + CHUNK)
    wait_out(0)
    wait_out(1)


_sc_lookup = functools.partial(
    pl.kernel,
    out_type=jax.ShapeDtypeStruct((B, D), jnp.float32),
    mesh=plsc.VectorSubcoreMesh(core_axis_name="c", subcore_axis_name="s"),
    scratch_types=[
        pltpu.VMEM((2, CHUNK), jnp.int32),
        pltpu.VMEM((2, ARG_NUM, CHUNK), jnp.int32),
        pltpu.VMEM((2, ARG_NUM * G, 128), jnp.int32),
        pltpu.VMEM((2, CHUNK, D), jnp.float32),
        pltpu.VMEM_SHARED((TBL_ROWS, D), jnp.float32),
        pltpu.SemaphoreType.DMA,
        pltpu.SemaphoreType.DMA,
        pltpu.SemaphoreType.DMA,
        pltpu.SemaphoreType.DMA,
        pltpu.SemaphoreType.DMA,
        pltpu.SemaphoreType.DMA,
        pltpu.SemaphoreType.DMA,
        pltpu.SemaphoreType.DMA,
    ],
)(_sc_body)


def kernel(commands, args, cmd_table, arg_table, W, b):
    flat_table = _fold(cmd_table, arg_table, W, b.reshape(1, D))
    argsT = args.reshape(B, ARG_NUM).T
    out = _sc_lookup(flat_table, commands.reshape(B), argsT)
    return out.reshape(N, S, D)
